# Initial kernel scaffold; baseline (speedup 1.0000x reference)
import numpy as np
import jax
import jax.numpy as jnp
from jax import lax
from jax.experimental import pallas as pl
from jax.experimental.pallas import tpu as pltpu
from jax.experimental.pallas import tpu_sc as plsc

VOCAB = 100000
EMB = 128
HID = 64
OUT = 256
B = 4096
L = 50

NC = 2
NS = 16
NW = NC * NS
BPW = B // NW
TPW = BPW * L
CH = 128
NCH = TPW // CH
BPC = B // NC

_DST = jnp.asarray(
    ((np.arange(B * L) // L) % BPC).astype(np.int32).reshape(NW, NCH, CH))


def _pool_body(tok_hbm, dst_hbm, table_hbm, zero_hbm, out_hbm,
               tok_v, dst_v, buf0, buf1, acc_sh, sem0, sem1):
    c = lax.axis_index("c")
    s = lax.axis_index("s")
    wid = c * NS + s

    pltpu.sync_copy(tok_hbm.at[wid], tok_v)
    pltpu.sync_copy(dst_hbm.at[wid], dst_v)

    pltpu.sync_copy(zero_hbm, buf0)
    pltpu.sync_copy(buf0, acc_sh.at[pl.ds(s * BPW, BPW)])

    bufs = (buf0, buf1)
    sems = (sem0, sem1)

    for b in range(2):
        pltpu.async_copy(table_hbm.at[tok_v.at[b]], bufs[b], sems[b])

    def body(i2, carry):
        g = i2 * 2
        for b in range(2):
            i = g + b
            pltpu.make_async_copy(
                table_hbm.at[tok_v.at[i]], bufs[b], sems[b]).wait()
            pltpu.sync_copy(bufs[b], acc_sh.at[dst_v.at[i]], add=True)
            pltpu.async_copy(table_hbm.at[tok_v.at[i + 2]], bufs[b], sems[b])
        return carry

    lax.fori_loop(0, (NCH - 2) // 2, body, 0)

    for b in range(2):
        i = NCH - 2 + b
        pltpu.make_async_copy(
            table_hbm.at[tok_v.at[i]], bufs[b], sems[b]).wait()
        pltpu.sync_copy(bufs[b], acc_sh.at[dst_v.at[i]], add=True)

    pltpu.sync_copy(acc_sh.at[pl.ds(s * BPW, BPW)], buf0)
    pltpu.sync_copy(buf0, out_hbm.at[pl.ds(wid * BPW, BPW)])


_pool = pl.kernel(
    _pool_body,
    mesh=plsc.VectorSubcoreMesh(core_axis_name="c", subcore_axis_name="s"),
    out_type=jax.ShapeDtypeStruct((B, EMB), jnp.float32),
    scratch_types=[
        pltpu.VMEM((NCH, CH), jnp.int32),
        pltpu.VMEM((NCH, CH), jnp.int32),
        pltpu.VMEM((CH, EMB), jnp.float32),
        pltpu.VMEM((CH, EMB), jnp.float32),
        pltpu.VMEM_SHARED((BPC, EMB), jnp.float32),
        pltpu.SemaphoreType.DMA,
        pltpu.SemaphoreType.DMA,
    ],
)

MB = 512


def _mlp_body(x_ref, w1_ref, b1_ref, w2_ref, b2_ref, w3_ref, b3_ref, o_ref):
    x = x_ref[...] * jnp.float32(1.0 / L)
    h = jnp.dot(x, w1_ref[...], preferred_element_type=jnp.float32)
    h = jnp.maximum(h + b1_ref[...], 0.0)
    h = jnp.dot(h, w2_ref[...], preferred_element_type=jnp.float32)
    h = jnp.maximum(h + b2_ref[...], 0.0)
    o = jnp.dot(h, w3_ref[...], preferred_element_type=jnp.float32)
    o_ref[...] = o + b3_ref[...]


_mlp = pl.pallas_call(
    _mlp_body,
    grid=(B // MB,),
    in_specs=[
        pl.BlockSpec((MB, EMB), lambda i: (i, 0)),
        pl.BlockSpec((EMB, HID), lambda i: (0, 0)),
        pl.BlockSpec((1, HID), lambda i: (0, 0)),
        pl.BlockSpec((HID, HID), lambda i: (0, 0)),
        pl.BlockSpec((1, HID), lambda i: (0, 0)),
        pl.BlockSpec((HID, OUT), lambda i: (0, 0)),
        pl.BlockSpec((1, OUT), lambda i: (0, 0)),
    ],
    out_specs=pl.BlockSpec((MB, OUT), lambda i: (i, 0)),
    out_shape=jax.ShapeDtypeStruct((B, OUT), jnp.float32),
)


def kernel(token_ids, emb_table, W1, b1, W2, b2, W3, b3):
    tok = token_ids.astype(jnp.int32).reshape(NW, NCH, CH)
    zero = jnp.zeros((BPW, EMB), jnp.float32)
    sums = _pool(tok, _DST, emb_table, zero)
    return _mlp(sums, W1, b1.reshape(1, HID), W2, b2.reshape(1, HID),
                W3, b3.reshape(1, OUT))


# speedup vs baseline: 9.0008x; 9.0008x over previous
import numpy as np
import jax
import jax.numpy as jnp
from jax import lax
from jax.experimental import pallas as pl
from jax.experimental.pallas import tpu as pltpu
from jax.experimental.pallas import tpu_sc as plsc

VOCAB = 100000
EMB = 128
HID = 64
OUT = 256
B = 4096
L = 50

NC = 2
NS = 16
NW = NC * NS
BPW = B // NW
TPW = BPW * L
CH = 128
NCH = TPW // CH
BPC = B // NC

_DST = ((np.arange(B * L) // L) % BPC).astype(np.int32).reshape(NW, NCH, CH)


def _pool_body(tok_hbm, dst_hbm, table_hbm, zero_hbm, out_hbm,
               tok_v, dst_v, buf0, buf1, acc_sh, sem0, sem1):
    c = lax.axis_index("c")
    s = lax.axis_index("s")
    wid = c * NS + s

    pltpu.sync_copy(tok_hbm.at[wid], tok_v)
    pltpu.sync_copy(dst_hbm.at[wid], dst_v)

    pltpu.sync_copy(zero_hbm, buf0)
    pltpu.sync_copy(buf0, acc_sh.at[pl.ds(s * BPW, BPW)])

    bufs = (buf0, buf1)
    sems = (sem0, sem1)

    for b in range(2):
        pltpu.async_copy(table_hbm.at[tok_v.at[b]], bufs[b], sems[b])

    def body(i2, carry):
        g = i2 * 2
        for b in range(2):
            i = g + b
            pltpu.make_async_copy(
                table_hbm.at[tok_v.at[i]], bufs[b], sems[b]).wait()
            pltpu.sync_copy(bufs[b], acc_sh.at[dst_v.at[i]], add=True)
            pltpu.async_copy(table_hbm.at[tok_v.at[i + 2]], bufs[b], sems[b])
        return carry

    lax.fori_loop(0, (NCH - 2) // 2, body, 0)

    for b in range(2):
        i = NCH - 2 + b
        pltpu.make_async_copy(
            table_hbm.at[tok_v.at[i]], bufs[b], sems[b]).wait()
        pltpu.sync_copy(bufs[b], acc_sh.at[dst_v.at[i]], add=True)

    pltpu.sync_copy(acc_sh.at[pl.ds(s * BPW, BPW)], buf0)
    pltpu.sync_copy(buf0, out_hbm.at[pl.ds(wid * BPW, BPW)])


_pool = pl.kernel(
    _pool_body,
    mesh=plsc.VectorSubcoreMesh(core_axis_name="c", subcore_axis_name="s"),
    out_type=jax.ShapeDtypeStruct((B, EMB), jnp.float32),
    scratch_types=[
        pltpu.VMEM((NCH, CH), jnp.int32),
        pltpu.VMEM((NCH, CH), jnp.int32),
        pltpu.VMEM((CH, EMB), jnp.float32),
        pltpu.VMEM((CH, EMB), jnp.float32),
        pltpu.VMEM_SHARED((BPC, EMB), jnp.float32),
        pltpu.SemaphoreType.DMA,
        pltpu.SemaphoreType.DMA,
    ],
)

MB = 512


def _mlp_body(x_ref, w1_ref, b1_ref, w2_ref, b2_ref, w3_ref, b3_ref, o_ref):
    x = x_ref[...] * jnp.float32(1.0 / L)
    h = jnp.dot(x, w1_ref[...], preferred_element_type=jnp.float32)
    h = jnp.maximum(h + b1_ref[...], 0.0)
    h = jnp.dot(h, w2_ref[...], preferred_element_type=jnp.float32)
    h = jnp.maximum(h + b2_ref[...], 0.0)
    o = jnp.dot(h, w3_ref[...], preferred_element_type=jnp.float32)
    o_ref[...] = o + b3_ref[...]


_mlp = pl.pallas_call(
    _mlp_body,
    grid=(B // MB,),
    in_specs=[
        pl.BlockSpec((MB, EMB), lambda i: (i, 0)),
        pl.BlockSpec((EMB, HID), lambda i: (0, 0)),
        pl.BlockSpec((1, HID), lambda i: (0, 0)),
        pl.BlockSpec((HID, HID), lambda i: (0, 0)),
        pl.BlockSpec((1, HID), lambda i: (0, 0)),
        pl.BlockSpec((HID, OUT), lambda i: (0, 0)),
        pl.BlockSpec((1, OUT), lambda i: (0, 0)),
    ],
    out_specs=pl.BlockSpec((MB, OUT), lambda i: (i, 0)),
    out_shape=jax.ShapeDtypeStruct((B, OUT), jnp.float32),
)


def kernel(token_ids, emb_table, W1, b1, W2, b2, W3, b3):
    tok = token_ids.astype(jnp.int32).reshape(NW, NCH, CH)
    zero = jnp.zeros((BPW, EMB), jnp.float32)
    sums = _pool(tok, jnp.asarray(_DST), emb_table, zero)
    return _mlp(sums, W1, b1.reshape(1, HID), W2, b2.reshape(1, HID),
                W3, b3.reshape(1, OUT))


# speedup vs baseline: 9.0926x; 1.0102x over previous
import numpy as np
import jax
import jax.numpy as jnp
from jax import lax
from jax.experimental import pallas as pl
from jax.experimental.pallas import tpu as pltpu
from jax.experimental.pallas import tpu_sc as plsc

VOCAB = 100000
EMB = 128
HID = 64
OUT = 256
B = 4096
L = 50

NC = 2
NS = 16
NW = NC * NS
BPW = B // NW
TPW = BPW * L
CH = 128
NCH = TPW // CH
BPC = B // NC

_DST = ((np.arange(B * L) // L) % BPC).astype(np.int32).reshape(NW, NCH, CH)


NBUF = 4


def _pool_body(tok_hbm, dst_hbm, table_hbm, zero_hbm, out_hbm,
               tok_v, dst_v, buf0, buf1, buf2, buf3, acc_sh,
               g0, g1, g2, g3, s0, s1, s2, s3):
    c = lax.axis_index("c")
    s = lax.axis_index("s")
    wid = c * NS + s
    bufs = (buf0, buf1, buf2, buf3)
    gsem = (g0, g1, g2, g3)
    ssem = (s0, s1, s2, s3)

    pltpu.sync_copy(tok_hbm.at[wid], tok_v)
    pltpu.sync_copy(dst_hbm.at[wid], dst_v)

    pltpu.sync_copy(zero_hbm, buf0)
    pltpu.sync_copy(buf0, acc_sh.at[pl.ds(s * BPW, BPW)])

    def _gather(i, b):
        return pltpu.async_copy(table_hbm.at[tok_v.at[i]], bufs[b], gsem[b])

    def _gather_wait(i, b):
        pltpu.make_async_copy(table_hbm.at[tok_v.at[i]], bufs[b],
                              gsem[b]).wait()

    def _scatter(i, b):
        return pltpu.async_copy(bufs[b], acc_sh.at[dst_v.at[i]], ssem[b],
                                add=True)

    for b in range(NBUF):
        _gather(b, b)

    def group(g, carry):
        base = g * NBUF
        hs = []
        for b in range(NBUF):
            _gather_wait(base + b, b)
            hs.append(_scatter(base + b, b))
        for b in range(NBUF):
            hs[b].wait()
            _gather(base + NBUF + b, b)
        return carry

    ngroups = NCH // NBUF - 1
    lax.fori_loop(0, ngroups, group, 0)

    base = ngroups * NBUF
    rem = NCH - NBUF - base
    hs = {}
    for b in range(NBUF):
        _gather_wait(base + b, b)
        hs[b] = _scatter(base + b, b)
    for b in range(rem):
        hs[b].wait()
        _gather(base + NBUF + b, b)
    for b in range(rem):
        _gather_wait(base + NBUF + b, b)
        hs[b] = _scatter(base + NBUF + b, b)
    for b in range(NBUF):
        hs[b].wait()

    pltpu.sync_copy(acc_sh.at[pl.ds(s * BPW, BPW)], buf0)
    pltpu.sync_copy(buf0, out_hbm.at[pl.ds(wid * BPW, BPW)])


_pool = pl.kernel(
    _pool_body,
    mesh=plsc.VectorSubcoreMesh(core_axis_name="c", subcore_axis_name="s"),
    out_type=jax.ShapeDtypeStruct((B, EMB), jnp.float32),
    scratch_types=[
        pltpu.VMEM((NCH, CH), jnp.int32),
        pltpu.VMEM((NCH, CH), jnp.int32),
        pltpu.VMEM((CH, EMB), jnp.float32),
        pltpu.VMEM((CH, EMB), jnp.float32),
        pltpu.VMEM((CH, EMB), jnp.float32),
        pltpu.VMEM((CH, EMB), jnp.float32),
        pltpu.VMEM_SHARED((BPC, EMB), jnp.float32),
        pltpu.SemaphoreType.DMA,
        pltpu.SemaphoreType.DMA,
        pltpu.SemaphoreType.DMA,
        pltpu.SemaphoreType.DMA,
        pltpu.SemaphoreType.DMA,
        pltpu.SemaphoreType.DMA,
        pltpu.SemaphoreType.DMA,
        pltpu.SemaphoreType.DMA,
    ],
)

MB = 512


def _mlp_body(x_ref, w1_ref, b1_ref, w2_ref, b2_ref, w3_ref, b3_ref, o_ref):
    x = x_ref[...] * jnp.float32(1.0 / L)
    h = jnp.dot(x, w1_ref[...], preferred_element_type=jnp.float32)
    h = jnp.maximum(h + b1_ref[...], 0.0)
    h = jnp.dot(h, w2_ref[...], preferred_element_type=jnp.float32)
    h = jnp.maximum(h + b2_ref[...], 0.0)
    o = jnp.dot(h, w3_ref[...], preferred_element_type=jnp.float32)
    o_ref[...] = o + b3_ref[...]


_mlp = pl.pallas_call(
    _mlp_body,
    grid=(B // MB,),
    in_specs=[
        pl.BlockSpec((MB, EMB), lambda i: (i, 0)),
        pl.BlockSpec((EMB, HID), lambda i: (0, 0)),
        pl.BlockSpec((1, HID), lambda i: (0, 0)),
        pl.BlockSpec((HID, HID), lambda i: (0, 0)),
        pl.BlockSpec((1, HID), lambda i: (0, 0)),
        pl.BlockSpec((HID, OUT), lambda i: (0, 0)),
        pl.BlockSpec((1, OUT), lambda i: (0, 0)),
    ],
    out_specs=pl.BlockSpec((MB, OUT), lambda i: (i, 0)),
    out_shape=jax.ShapeDtypeStruct((B, OUT), jnp.float32),
)


def kernel(token_ids, emb_table, W1, b1, W2, b2, W3, b3):
    tok = token_ids.astype(jnp.int32).reshape(NW, NCH, CH)
    zero = jnp.zeros((BPW, EMB), jnp.float32)
    sums = _pool(tok, jnp.asarray(_DST), emb_table, zero)
    return _mlp(sums, W1, b1.reshape(1, HID), W2, b2.reshape(1, HID),
                W3, b3.reshape(1, OUT))


# speedup vs baseline: 13.4797x; 1.4825x over previous
import numpy as np
import jax
import jax.numpy as jnp
from jax import lax
from jax.experimental import pallas as pl
from jax.experimental.pallas import tpu as pltpu
from jax.experimental.pallas import tpu_sc as plsc

VOCAB = 100000
EMB = 128
HID = 64
OUT = 256
B = 4096
L = 50

NC = 2
NS = 16
NW = NC * NS
BPW = B // NW


def _pool_body(tok_hbm, table_hbm, zero_hbm, out_hbm, tok_v, acc, sem):
    c = lax.axis_index("c")
    s = lax.axis_index("s")
    wid = c * NS + s

    pltpu.sync_copy(tok_hbm.at[wid], tok_v)
    pltpu.sync_copy(zero_hbm, acc)

    def fire(t, carry):
        pltpu.async_copy(table_hbm.at[tok_v.at[t]], acc, sem, add=True)
        return carry

    lax.fori_loop(0, L, fire, 0)

    def drain(t, carry):
        pltpu.make_async_copy(table_hbm.at[tok_v.at[0]], acc, sem).wait()
        return carry

    lax.fori_loop(0, L, drain, 0)

    pltpu.sync_copy(acc, out_hbm.at[pl.ds(wid * BPW, BPW)])


_pool = pl.kernel(
    _pool_body,
    mesh=plsc.VectorSubcoreMesh(core_axis_name="c", subcore_axis_name="s"),
    out_type=jax.ShapeDtypeStruct((B, EMB), jnp.float32),
    scratch_types=[
        pltpu.VMEM((L, BPW), jnp.int32),
        pltpu.VMEM((BPW, EMB), jnp.float32),
        pltpu.SemaphoreType.DMA,
    ],
)

MB = 512


def _mlp_body(x_ref, w1_ref, b1_ref, w2_ref, b2_ref, w3_ref, b3_ref, o_ref):
    x = x_ref[...] * jnp.float32(1.0 / L)
    h = jnp.dot(x, w1_ref[...], preferred_element_type=jnp.float32)
    h = jnp.maximum(h + b1_ref[...], 0.0)
    h = jnp.dot(h, w2_ref[...], preferred_element_type=jnp.float32)
    h = jnp.maximum(h + b2_ref[...], 0.0)
    o = jnp.dot(h, w3_ref[...], preferred_element_type=jnp.float32)
    o_ref[...] = o + b3_ref[...]


_mlp = pl.pallas_call(
    _mlp_body,
    grid=(B // MB,),
    in_specs=[
        pl.BlockSpec((MB, EMB), lambda i: (i, 0)),
        pl.BlockSpec((EMB, HID), lambda i: (0, 0)),
        pl.BlockSpec((1, HID), lambda i: (0, 0)),
        pl.BlockSpec((HID, HID), lambda i: (0, 0)),
        pl.BlockSpec((1, HID), lambda i: (0, 0)),
        pl.BlockSpec((HID, OUT), lambda i: (0, 0)),
        pl.BlockSpec((1, OUT), lambda i: (0, 0)),
    ],
    out_specs=pl.BlockSpec((MB, OUT), lambda i: (i, 0)),
    out_shape=jax.ShapeDtypeStruct((B, OUT), jnp.float32),
)


def kernel(token_ids, emb_table, W1, b1, W2, b2, W3, b3):
    tok = token_ids.astype(jnp.int32).reshape(NW, BPW, L).transpose(0, 2, 1)
    zero = jnp.zeros((BPW, EMB), jnp.float32)
    sums = _pool(tok, emb_table, zero)
    return _mlp(sums, W1, b1.reshape(1, HID), W2, b2.reshape(1, HID),
                W3, b3.reshape(1, OUT))
